# Initial kernel scaffold; baseline (speedup 1.0000x reference)
import jax
import jax.numpy as jnp
from jax.experimental import pallas as pl


def kernel(source_feat, target_feat, prototypes):
    raise NotImplementedError("write your pallas kernel here")


# speedup vs baseline: 2.2269x; 2.2269x over previous
import functools
import jax
import jax.numpy as jnp
from jax.experimental import pallas as pl
from jax.experimental.pallas import tpu as pltpu

FEAT = 64
NPROT = 8
MOM = 0.9
N_ROWS = 16384
BLK = 2048
NBLK = N_ROWS // BLK


def _row_normalize(x):
    n = jnp.sqrt(jnp.sum(x * x, axis=-1, keepdims=True))
    return x / jnp.maximum(n, 1e-12)


def _src_body(src_ref, protos_ref, out_ref, sums_ref, counts_ref):
    i = pl.program_id(0)

    @pl.when(i == 0)
    def _init():
        sums_ref[...] = jnp.zeros_like(sums_ref)
        counts_ref[...] = jnp.zeros_like(counts_ref)

    x = src_ref[...]
    xn = _row_normalize(x)
    protos = protos_ref[...]
    dots = jax.lax.dot_general(
        xn, protos, (((1,), (1,)), ((), ())),
        preferred_element_type=jnp.float32,
        precision=jax.lax.Precision.HIGHEST,
    )
    assign = jnp.argmax(dots, axis=1)
    one_hot = (assign[:, None] == jax.lax.broadcasted_iota(jnp.int32, (1, NPROT), 1)
               ).astype(jnp.float32)
    psums = jax.lax.dot_general(
        one_hot, xn, (((0,), (0,)), ((), ())),
        preferred_element_type=jnp.float32,
        precision=jax.lax.Precision.HIGHEST,
    )
    sums_ref[...] += psums
    counts_ref[...] += jnp.sum(one_hot, axis=0, keepdims=True)

    @pl.when(i == NBLK - 1)
    def _finish():
        counts_row = counts_ref[...]
        r = jax.lax.broadcasted_iota(jnp.int32, (NPROT, NPROT), 0)
        c = jax.lax.broadcasted_iota(jnp.int32, (NPROT, NPROT), 1)
        counts_col = jnp.sum(
            jnp.where(r == c, jnp.broadcast_to(counts_row, (NPROT, NPROT)), 0.0),
            axis=1, keepdims=True)
        cm = sums_ref[...] / jnp.maximum(counts_col, 1.0)
        cmn = _row_normalize(cm)
        upd = MOM * protos_ref[...] + (1.0 - MOM) * cmn
        upd = jnp.where(counts_col > 0.0, upd, protos_ref[...])
        out_ref[...] = _row_normalize(upd)


def _tgt_body(tgt_ref, protos_ref, out_ref, acc_ref):
    i = pl.program_id(0)

    @pl.when(i == 0)
    def _init():
        acc_ref[0, 0] = 0.0

    t = tgt_ref[...]
    tn = _row_normalize(t)
    cos = jax.lax.dot_general(
        tn, protos_ref[...], (((1,), (1,)), ((), ())),
        preferred_element_type=jnp.float32,
        precision=jax.lax.Precision.HIGHEST,
    )
    m = jnp.max(cos, axis=1, keepdims=True)
    acc_ref[0, 0] += jnp.sum(1.0 - m)

    @pl.when(i == NBLK - 1)
    def _finish():
        out_ref[0, 0] = acc_ref[0, 0] / N_ROWS


@jax.jit
def kernel(source_feat, target_feat, prototypes):
    new_protos = pl.pallas_call(
        _src_body,
        grid=(NBLK,),
        in_specs=[
            pl.BlockSpec((BLK, FEAT), lambda i: (i, 0)),
            pl.BlockSpec((NPROT, FEAT), lambda i: (0, 0)),
        ],
        out_specs=pl.BlockSpec((NPROT, FEAT), lambda i: (0, 0)),
        out_shape=jax.ShapeDtypeStruct((NPROT, FEAT), jnp.float32),
        scratch_shapes=[
            pltpu.VMEM((NPROT, FEAT), jnp.float32),
            pltpu.VMEM((1, NPROT), jnp.float32),
        ],
    )(source_feat, prototypes)

    loss = pl.pallas_call(
        _tgt_body,
        grid=(NBLK,),
        in_specs=[
            pl.BlockSpec((BLK, FEAT), lambda i: (i, 0)),
            pl.BlockSpec((NPROT, FEAT), lambda i: (0, 0)),
        ],
        out_specs=pl.BlockSpec((1, 1), lambda i: (0, 0), memory_space=pltpu.SMEM),
        out_shape=jax.ShapeDtypeStruct((1, 1), jnp.float32),
        scratch_shapes=[pltpu.SMEM((1, 1), jnp.float32)],
    )(target_feat, new_protos)

    return loss[0, 0]
